# Initial kernel scaffold; baseline (speedup 1.0000x reference)
import jax
import jax.numpy as jnp
from jax.experimental import pallas as pl


def kernel(x, importance, w0, b0, w1, b1):
    raise NotImplementedError("write your pallas kernel here")


# speedup vs baseline: 6.7942x; 6.7942x over previous
import functools

import jax
import jax.numpy as jnp
from jax.experimental import pallas as pl
from jax.experimental.pallas import tpu as pltpu


def _mask_body(keep, imp_row_ref, imp_col_ref, mask_ref):
    d_total = imp_row_ref.shape[1]
    imp_row = imp_row_ref[:, :]
    d_ids = jax.lax.broadcasted_iota(jnp.int32, (1, d_total), 1)
    rank = jnp.zeros((1, d_total), jnp.float32)
    chunk = 256
    for c in range(d_total // chunk):
        col = imp_col_ref[pl.ds(c * chunk, chunk), :]
        e_ids = c * chunk + jax.lax.broadcasted_iota(
            jnp.int32, (chunk, 1), 0)
        gt = col > imp_row
        tie = (col == imp_row) & (e_ids < d_ids)
        rank += jnp.sum((gt | tie).astype(jnp.float32), axis=0,
                        keepdims=True)
    mask_ref[:, :] = (rank < float(keep)).astype(jnp.float32)


def _select_body(mask_ref, p_ref, x_ref, o_ref):
    x = x_ref[:, :]
    w0 = p_ref[0, 0]
    b0 = p_ref[0, 1]
    w1 = p_ref[0, 2]
    b1 = p_ref[0, 3]
    y = x * w0 + b0
    y = y * jax.nn.sigmoid(y)
    y = y * w1 + b1
    y = y * jax.nn.sigmoid(y)
    m = mask_ref[:, :] > 0.0
    o_ref[:, :] = jnp.where(m, y, x)


def _topk_mask(importance, keep):
    d_total = importance.shape[0]
    return pl.pallas_call(
        functools.partial(_mask_body, keep),
        out_shape=jax.ShapeDtypeStruct((1, d_total), jnp.float32),
    )(importance.reshape(1, d_total), importance.reshape(d_total, 1))


def kernel(x, importance, w0, b0, w1, b1):
    b_sz, t_sz, d_sz = x.shape
    keep = max(1, int(d_sz * 0.25))
    rows_total = b_sz * t_sz
    xf = x.reshape(rows_total, d_sz)
    params = jnp.stack(
        [w0[0, 0], b0[0], w1[0, 0], b1[0]]).reshape(1, 4)

    mask = _topk_mask(importance, keep)

    rows = 512
    grid = (rows_total // rows,)
    out = pl.pallas_call(
        _select_body,
        grid=grid,
        in_specs=[
            pl.BlockSpec((1, d_sz), lambda i: (0, 0)),
            pl.BlockSpec(memory_space=pltpu.SMEM),
            pl.BlockSpec((rows, d_sz), lambda i: (i, 0)),
        ],
        out_specs=pl.BlockSpec((rows, d_sz), lambda i: (i, 0)),
        out_shape=jax.ShapeDtypeStruct((rows_total, d_sz), jnp.float32),
        compiler_params=pltpu.CompilerParams(
            dimension_semantics=("arbitrary",)),
    )(mask, params, xf)
    return out.reshape(b_sz, t_sz, d_sz)
